# Initial kernel scaffold; baseline (speedup 1.0000x reference)
import jax
import jax.numpy as jnp
from jax.experimental import pallas as pl


def kernel(x, edge_index, W, b):
    raise NotImplementedError("write your pallas kernel here")


# speedup vs baseline: 19.8881x; 19.8881x over previous
import functools

import jax
import jax.numpy as jnp
from jax import lax
from jax.experimental import pallas as pl
from jax.experimental.pallas import tpu as pltpu
from jax.experimental.pallas import tpu_sc as plsc

N = 10000
E = 320000
D = 128

NC = 2
NS = 16
NW = NC * NS
EPW = E // NW
K = 80
NCHUNK = EPW // K
ACC_ROWS = 10240
ZROWS = 128
ROWS_PT = N // NS

BR = 400
GRID = N // BR

_mesh = plsc.VectorSubcoreMesh(core_axis_name="c", subcore_axis_name="s")


@functools.partial(
    pl.kernel,
    mesh=_mesh,
    compiler_params=pltpu.CompilerParams(needs_layout_passes=False),
    out_type=jax.ShapeDtypeStruct((NW, N), jnp.float32),
    scratch_types=[
        pltpu.VMEM((N,), jnp.float32),
        pltpu.VMEM((EPW,), jnp.int32),
        pltpu.VMEM((EPW,), jnp.int32),
    ],
)
def _deg_kernel(row_hbm, col_hbm, hist_hbm, hist_v, row_v, col_v):
    wid = lax.axis_index("s") * NC + lax.axis_index("c")
    base = wid * EPW
    pltpu.sync_copy(row_hbm.at[pl.ds(base, EPW)], row_v)
    pltpu.sync_copy(col_hbm.at[pl.ds(base, EPW)], col_v)

    zv = jnp.zeros((16,), jnp.float32)

    def zbody(i, t):
        hist_v[pl.ds(i * 16, 16)] = zv
        return t

    lax.fori_loop(0, N // 16, zbody, 0)

    ones = jnp.ones((16,), jnp.float32)
    zero = jnp.zeros((16,), jnp.float32)

    def ebody(i, t):
        r = row_v[pl.ds(i * 16, 16)]
        c = col_v[pl.ds(i * 16, 16)]
        val = jnp.where(r == c, zero, ones)
        plsc.addupdate_scatter(hist_v, [r], val)
        return t

    lax.fori_loop(0, EPW // 16, ebody, 0)
    pltpu.sync_copy(hist_v, hist_hbm.at[wid])


def _norm_body(x_ref, w_ref, hist_ref, m_ref):
    h = jnp.dot(x_ref[...], w_ref[...], preferred_element_type=jnp.float32)
    deg = jnp.sum(hist_ref[...], axis=1) + 1.0
    dinv = lax.rsqrt(deg)
    m_ref[...] = h * dinv[:, None]


@functools.partial(
    pl.kernel,
    mesh=_mesh,
    compiler_params=pltpu.CompilerParams(needs_layout_passes=False),
    out_type=jax.ShapeDtypeStruct((NC, ACC_ROWS, D), jnp.float32),
    scratch_types=[
        pltpu.VMEM_SHARED((ACC_ROWS, D), jnp.float32),
        pltpu.VMEM((K,), jnp.int32),
        pltpu.VMEM((K,), jnp.int32),
        pltpu.VMEM((K, D), jnp.float32),
        pltpu.VMEM((ZROWS, D), jnp.float32),
        pltpu.SemaphoreType.DMA,
    ],
)
def _edge_kernel(m_hbm, row_hbm, col_hbm, outp_hbm,
                 acc_s, row_v, col_v, gbuf, zbuf, sem):
    cid = lax.axis_index("c")
    sid = lax.axis_index("s")
    wid = sid * NC + cid

    zv = jnp.zeros((16,), jnp.float32)

    def zb(i, t):
        zbuf[i // 8, pl.ds((i % 8) * 16, 16)] = zv
        return t

    lax.fori_loop(0, ZROWS * (D // 16), zb, 0)

    def zacc(i, t):
        pltpu.sync_copy(zbuf, acc_s.at[pl.ds((sid * 5 + i) * ZROWS, ZROWS)])
        return t

    lax.fori_loop(0, ACC_ROWS // (NS * ZROWS), zacc, 0)
    plsc.subcore_barrier()

    trash = jnp.full((16,), N, jnp.int32)
    base = wid * EPW

    def chunk(ci, t):
        off = base + ci * K
        pltpu.sync_copy(row_hbm.at[pl.ds(off, K)], row_v)
        pltpu.sync_copy(col_hbm.at[pl.ds(off, K)], col_v)

        def adj(j, t2):
            r = row_v[pl.ds(j * 16, 16)]
            c = col_v[pl.ds(j * 16, 16)]
            col_v[pl.ds(j * 16, 16)] = jnp.where(r == c, trash, c)
            return t2

        lax.fori_loop(0, K // 16, adj, 0)
        pltpu.async_copy(m_hbm.at[row_v], gbuf, sem).wait()
        pltpu.sync_copy(gbuf, acc_s.at[col_v], add=True)
        return t

    lax.fori_loop(0, NCHUNK, chunk, 0)
    plsc.subcore_barrier()
    rpt = ACC_ROWS // NS
    pltpu.sync_copy(acc_s.at[pl.ds(sid * rpt, rpt)],
                    outp_hbm.at[cid, pl.ds(sid * rpt, rpt)])


def _final_body(p_ref, m_ref, hist_ref, b_ref, o_ref):
    deg = jnp.sum(hist_ref[...], axis=1) + 1.0
    dinv = lax.rsqrt(deg)
    s = p_ref[0] + p_ref[1] + m_ref[...]
    o_ref[...] = dinv[:, None] * s + b_ref[...]


def kernel(x, edge_index, W, b):
    row = edge_index[0].astype(jnp.int32)
    col = edge_index[1].astype(jnp.int32)

    hist = _deg_kernel(row, col).T

    m = pl.pallas_call(
        _norm_body,
        grid=(GRID,),
        in_specs=[
            pl.BlockSpec((BR, D), lambda r: (r, 0)),
            pl.BlockSpec((D, D), lambda r: (0, 0)),
            pl.BlockSpec((BR, NW), lambda r: (r, 0)),
        ],
        out_specs=pl.BlockSpec((BR, D), lambda r: (r, 0)),
        out_shape=jax.ShapeDtypeStruct((N, D), jnp.float32),
    )(x, W, hist)

    partials = _edge_kernel(m, row, col)

    out = pl.pallas_call(
        _final_body,
        grid=(GRID,),
        in_specs=[
            pl.BlockSpec((NC, BR, D), lambda r: (0, r, 0)),
            pl.BlockSpec((BR, D), lambda r: (r, 0)),
            pl.BlockSpec((BR, NW), lambda r: (r, 0)),
            pl.BlockSpec((1, D), lambda r: (0, 0)),
        ],
        out_specs=pl.BlockSpec((BR, D), lambda r: (r, 0)),
        out_shape=jax.ShapeDtypeStruct((N, D), jnp.float32),
    )(partials, m, hist, b[None, :])

    return out
